# baseline (speedup 1.0000x reference)
import functools

import jax
import jax.numpy as jnp
from jax import lax
from jax.experimental import pallas as pl
from jax.experimental.pallas import tpu as pltpu
from jax.experimental.pallas import tpu_sc as plsc

B = 32
P = 784
NCHOICE = 2
OUTW = P * NCHOICE
LANES = 16


def _tile_body(fakes_hbm, reals_hbm, idx_hbm, out_hbm,
               buf, idxbuf, outbuf, sem_f, sem_r, sem_i):
    wid = lax.axis_index("s") * 2 + lax.axis_index("c")
    cp_f = pltpu.async_copy(fakes_hbm.at[wid], buf.at[0], sem_f)
    cp_r = pltpu.async_copy(reals_hbm.at[wid], buf.at[1], sem_r)
    cp_i = pltpu.async_copy(idx_hbm, idxbuf, sem_i)

    lanes = lax.iota(jnp.int32, LANES)
    jmod = lanes & 1
    half = lanes >> 1

    cp_i.wait()
    sel = plsc.load_gather(idxbuf, [2 * wid + jmod])
    cp_f.wait()
    cp_r.wait()

    for k in range(P // 8):
        pvec = k * 8 + half
        vals = plsc.load_gather(buf, [sel, pvec])
        outbuf[pl.ds(k * LANES, LANES)] = vals

    pltpu.sync_copy(outbuf, out_hbm.at[wid])


_sc_call = functools.partial(
    pl.kernel,
    out_type=jax.ShapeDtypeStruct((B, OUTW), jnp.float32),
    mesh=plsc.VectorSubcoreMesh(core_axis_name="c", subcore_axis_name="s"),
    scratch_types=[
        pltpu.VMEM((NCHOICE, P), jnp.float32),
        pltpu.VMEM((B * NCHOICE,), jnp.int32),
        pltpu.VMEM((OUTW,), jnp.float32),
        pltpu.SemaphoreType.DMA,
        pltpu.SemaphoreType.DMA,
        pltpu.SemaphoreType.DMA,
    ],
    compiler_params=pltpu.CompilerParams(needs_layout_passes=False),
)(_tile_body)


@jax.jit
def kernel(reals, fakes, shuffle_indices):
    fakes2 = fakes.astype(jnp.float32).reshape(B, P)
    reals2 = reals.astype(jnp.float32).reshape(B, P)
    idx = shuffle_indices.astype(jnp.int32).reshape(B * NCHOICE)
    out = _sc_call(fakes2, reals2, idx)
    return out.reshape(B, 28, 28, NCHOICE, 1)
